# speedup vs baseline: 1.2789x; 1.2789x over previous
import functools

import jax
import jax.numpy as jnp
from jax import lax
from jax.experimental import pallas as pl
from jax.experimental.pallas import tpu as pltpu
from jax.experimental.pallas import tpu_sc as plsc

_NC = 2
_NS = 16
_NW = _NC * _NS
_CH = 128


@jax.jit
def _embed(xf, table):
    n_rows, ch = xf.shape
    v, d = table.shape
    n_chunks = n_rows // _NW

    mesh = plsc.VectorSubcoreMesh(core_axis_name="c", subcore_axis_name="s")

    @functools.partial(
        pl.kernel,
        out_type=jax.ShapeDtypeStruct((n_rows * ch, d), jnp.float32),
        mesh=mesh,
        scratch_types=[
            pltpu.VMEM((n_chunks, ch), jnp.int32),
            pltpu.VMEM((ch, d), jnp.float32),
            pltpu.SemaphoreType.DMA,
        ],
    )
    def emb(x_hbm, table_hbm, out_hbm, idx_v, rows_v, sem):
        wid = lax.axis_index("s") * _NC + lax.axis_index("c")
        row0 = wid * n_chunks
        pltpu.sync_copy(x_hbm.at[pl.ds(row0, n_chunks)], idx_v)

        @pl.loop(0, n_chunks)
        def _(j):
            pltpu.async_copy(table_hbm.at[idx_v.at[j]], rows_v, sem).wait()
            pltpu.sync_copy(rows_v, out_hbm.at[pl.ds((row0 + j) * ch, ch)])

    return emb(xf, table)


def kernel(x, table):
    b, l = x.shape
    _, d = table.shape
    xf = x.reshape(b * l // _CH, _CH).astype(jnp.int32)
    out = _embed(xf, table)
    return out.reshape(b, l, d)


# speedup vs baseline: 1.8603x; 1.4546x over previous
import functools

import jax
import jax.numpy as jnp
from jax import lax
from jax.experimental import pallas as pl
from jax.experimental.pallas import tpu as pltpu
from jax.experimental.pallas import tpu_sc as plsc

_NC = 2
_NS = 16
_NW = _NC * _NS
_CH = 128


@jax.jit
def _embed(xf, table):
    n_rows, ch = xf.shape
    v, d = table.shape
    n_chunks = n_rows // _NW

    mesh = plsc.VectorSubcoreMesh(core_axis_name="c", subcore_axis_name="s")
    nb = 4

    @functools.partial(
        pl.kernel,
        out_type=jax.ShapeDtypeStruct((n_rows * ch, d), jnp.float32),
        mesh=mesh,
        scratch_types=[
            pltpu.VMEM((n_chunks, ch), jnp.int32),
            pltpu.VMEM((nb, ch, d), jnp.float32),
            [pltpu.SemaphoreType.DMA] * nb,
            [pltpu.SemaphoreType.DMA] * nb,
        ],
    )
    def emb(x_hbm, table_hbm, out_hbm, idx_v, rows_v, sem_g, sem_s):
        wid = lax.axis_index("s") * _NC + lax.axis_index("c")
        row0 = wid * n_chunks
        pltpu.sync_copy(x_hbm.at[pl.ds(row0, n_chunks)], idx_v)

        def gather_start(j, b):
            pltpu.async_copy(table_hbm.at[idx_v.at[j]], rows_v.at[b], sem_g[b])

        def gather_wait(j, b):
            pltpu.make_async_copy(
                table_hbm.at[idx_v.at[j]], rows_v.at[b], sem_g[b]
            ).wait()

        def out_start(j, b):
            pltpu.async_copy(
                rows_v.at[b], out_hbm.at[pl.ds((row0 + j) * ch, ch)], sem_s[b]
            )

        def out_wait(j, b):
            pltpu.make_async_copy(
                rows_v.at[b], out_hbm.at[pl.ds((row0 + j) * ch, ch)], sem_s[b]
            ).wait()

        gather_start(0, 0)
        gather_start(1, 1)

        @pl.loop(0, n_chunks // nb)
        def _(g):
            for b in range(nb):
                j = g * nb + b
                gather_wait(j, b)
                out_start(j, b)

                @pl.when(j >= 2)
                def _():
                    out_wait(j - 2, (b + 2) % nb)

                @pl.when(j + 2 < n_chunks)
                def _():
                    gather_start(j + 2, (b + 2) % nb)

        out_wait(n_chunks - 2, (n_chunks - 2) % nb)
        out_wait(n_chunks - 1, (n_chunks - 1) % nb)

    return emb(xf, table)


def kernel(x, table):
    b, l = x.shape
    _, d = table.shape
    xf = x.reshape(b * l // _CH, _CH).astype(jnp.int32)
    out = _embed(xf, table)
    return out.reshape(b, l, d)


# speedup vs baseline: 1.8633x; 1.0016x over previous
import functools

import jax
import jax.numpy as jnp
from jax import lax
from jax.experimental import pallas as pl
from jax.experimental.pallas import tpu as pltpu
from jax.experimental.pallas import tpu_sc as plsc

_NC = 2
_NS = 16
_NW = _NC * _NS
_CH = 128


@jax.jit
def _embed(xf, table):
    n_rows, ch = xf.shape
    v, d = table.shape
    n_chunks = n_rows // _NW

    mesh = plsc.VectorSubcoreMesh(core_axis_name="c", subcore_axis_name="s")
    nb = 5

    @functools.partial(
        pl.kernel,
        out_type=jax.ShapeDtypeStruct((n_rows * ch, d), jnp.float32),
        mesh=mesh,
        scratch_types=[
            pltpu.VMEM((n_chunks, ch), jnp.int32),
            pltpu.VMEM((nb, ch, d), jnp.float32),
            [pltpu.SemaphoreType.DMA] * nb,
            [pltpu.SemaphoreType.DMA] * nb,
        ],
    )
    def emb(x_hbm, table_hbm, out_hbm, idx_v, rows_v, sem_g, sem_s):
        wid = lax.axis_index("s") * _NC + lax.axis_index("c")
        row0 = wid * n_chunks
        pltpu.sync_copy(x_hbm.at[pl.ds(row0, n_chunks)], idx_v)

        def gather_start(j, b):
            pltpu.async_copy(table_hbm.at[idx_v.at[j]], rows_v.at[b], sem_g[b])

        def gather_wait(j, b):
            pltpu.make_async_copy(
                table_hbm.at[idx_v.at[j]], rows_v.at[b], sem_g[b]
            ).wait()

        def out_start(j, b):
            pltpu.async_copy(
                rows_v.at[b], out_hbm.at[pl.ds((row0 + j) * ch, ch)], sem_s[b]
            )

        def out_wait(j, b):
            pltpu.make_async_copy(
                rows_v.at[b], out_hbm.at[pl.ds((row0 + j) * ch, ch)], sem_s[b]
            ).wait()

        gather_start(0, 0)
        gather_start(1, 1)
        gather_start(2, 2)

        @pl.loop(0, n_chunks // nb)
        def _(g):
            for b in range(nb):
                j = g * nb + b
                gather_wait(j, b)
                out_start(j, b)

                @pl.when(j >= 2)
                def _():
                    out_wait(j - 2, (b + 3) % nb)

                @pl.when(j + 3 < n_chunks)
                def _():
                    gather_start(j + 3, (b + 3) % nb)

        out_wait(n_chunks - 2, (n_chunks - 2) % nb)
        out_wait(n_chunks - 1, (n_chunks - 1) % nb)

    return emb(xf, table)


def kernel(x, table):
    b, l = x.shape
    _, d = table.shape
    xf = x.reshape(b * l // _CH, _CH).astype(jnp.int32)
    out = _embed(xf, table)
    return out.reshape(b, l, d)
